# baseline (speedup 1.0000x reference)
import jax
import jax.numpy as jnp
from jax import lax
from jax.experimental import pallas as pl
from jax.experimental.pallas import tpu as pltpu
from jax.experimental.pallas import tpu_sc as plsc

D_EMB = 16
N_IDX = 10
N_GAR = 21
D_OUT = 10
BATCH = 16384

_info = plsc.get_sparse_core_info()
NC, NS, L = _info.num_cores, _info.num_subcores, _info.num_lanes
NW = NC * NS
BPW = BATCH // NW
CHUNK = 128
N_CHUNK = BPW // CHUNK
N_BLK = BPW // L


def _tower_body(ids_hbm, ig_hbm, gg_hbm, emb_hbm, wbc_hbm, wot_hbm, out_hbm,
                idx_v, rows_v, rows_t, ig_v, gg_v, wot_v, wbc_v, out_v, sem):
    wid = lax.axis_index("s") * NC + lax.axis_index("c")
    base = wid * BPW

    pltpu.sync_copy(ids_hbm.at[wid], idx_v)
    gathers = [
        pltpu.async_copy(emb_hbm.at[idx_v.at[j]],
                         rows_v.at[pl.ds(j * CHUNK, CHUNK)], sem)
        for j in range(N_CHUNK)
    ]
    pltpu.sync_copy(ig_hbm.at[pl.ds(base, BPW)], ig_v)
    pltpu.sync_copy(gg_hbm.at[pl.ds(base, BPW)], gg_v)
    pltpu.sync_copy(wot_hbm, wot_v)
    pltpu.sync_copy(wbc_hbm, wbc_v)
    for g in gathers:
        g.wait()

    iota = lax.iota(jnp.int32, L)
    tpose = iota * BPW

    def blk(j, carry):
        r0 = j * L
        for di in range(L):
            e = jnp.maximum(rows_v[r0 + di, :], 0.0)
            plsc.store_scatter(rows_t, [tpose + (r0 + di)], e)
        cols = [rows_t[pl.ds(d * BPW + r0, L)] for d in range(D_EMB)]
        ig_blk = ig_v[pl.ds(r0, L)] * D_OUT
        gg_blk = (gg_v[pl.ds(r0, L)] + N_IDX) * D_OUT
        obase = (iota + r0) * D_OUT
        for k in range(D_OUT):
            acc = plsc.load_gather(wot_v, [ig_blk + k])
            acc = acc + plsc.load_gather(wot_v, [gg_blk + k])
            for d in range(D_EMB):
                acc = acc + cols[d] * wbc_v[pl.ds((k * D_EMB + d) * L, L)]
            plsc.store_scatter(out_v, [obase + k], acc)
        return carry

    lax.fori_loop(0, N_BLK, blk, 0)
    pltpu.sync_copy(out_v, out_hbm.at[pl.ds(base * D_OUT, BPW * D_OUT)])


_run = pl.kernel(
    _tower_body,
    out_type=jax.ShapeDtypeStruct((BATCH * D_OUT,), jnp.float32),
    mesh=plsc.VectorSubcoreMesh(core_axis_name="c", subcore_axis_name="s"),
    compiler_params=pltpu.CompilerParams(needs_layout_passes=False,
                                         use_tc_tiling_on_sc=False),
    scratch_types=[
        pltpu.VMEM((N_CHUNK, CHUNK), jnp.int32),
        pltpu.VMEM((BPW, D_EMB), jnp.float32),
        pltpu.VMEM((D_EMB * BPW,), jnp.float32),
        pltpu.VMEM((BPW,), jnp.int32),
        pltpu.VMEM((BPW,), jnp.int32),
        pltpu.VMEM(((N_IDX + N_GAR) * D_OUT,), jnp.float32),
        pltpu.VMEM((D_OUT * D_EMB * L,), jnp.float32),
        pltpu.VMEM((BPW * D_OUT,), jnp.float32),
        pltpu.SemaphoreType.DMA,
    ],
)


def kernel(item_ids, index_group_names, garment_group_names, emb_table, W, b):
    ids = item_ids.astype(jnp.int32).reshape(NW, N_CHUNK, CHUNK)
    wbc = jnp.repeat(W[:, :D_EMB].reshape(-1), L)
    wot = jnp.concatenate(
        [W[:, D_EMB:D_EMB + N_IDX].T, W[:, D_EMB + N_IDX:].T + b[None, :]],
        axis=0,
    ).reshape(-1)
    out = _run(ids, index_group_names.astype(jnp.int32),
               garment_group_names.astype(jnp.int32), emb_table, wbc, wot)
    return out.reshape(BATCH, D_OUT)


# speedup vs baseline: 1.0142x; 1.0142x over previous
import jax
import jax.numpy as jnp
from jax import lax
from jax.experimental import pallas as pl
from jax.experimental.pallas import tpu as pltpu
from jax.experimental.pallas import tpu_sc as plsc

D_EMB = 16
N_IDX = 10
N_GAR = 21
D_OUT = 10
BATCH = 16384

_info = plsc.get_sparse_core_info()
NC, NS, L = _info.num_cores, _info.num_subcores, _info.num_lanes
NW = NC * NS
BPW = BATCH // NW
CHUNK = 128
N_CHUNK = BPW // CHUNK
N_BLK = BPW // L


def _tower_body(ids_hbm, ig_hbm, gg_hbm, emb_hbm, wbc_hbm, wot_hbm, out_hbm,
                idx_v, rows_v, rows_t, ig_v, gg_v, wot_v, wbc_v, out_v, sem):
    wid = lax.axis_index("s") * NC + lax.axis_index("c")
    base = wid * BPW

    pltpu.sync_copy(ids_hbm.at[wid], idx_v)
    gathers = [
        pltpu.async_copy(emb_hbm.at[idx_v.at[j]],
                         rows_v.at[pl.ds(j * CHUNK, CHUNK)], sem)
        for j in range(N_CHUNK)
    ]
    pltpu.sync_copy(ig_hbm.at[pl.ds(base, BPW)], ig_v)
    pltpu.sync_copy(gg_hbm.at[pl.ds(base, BPW)], gg_v)
    pltpu.sync_copy(wot_hbm, wot_v)
    pltpu.sync_copy(wbc_hbm, wbc_v)
    for g in gathers:
        g.wait()

    iota = lax.iota(jnp.int32, L)
    tpose = iota * BPW

    @plsc.parallel_loop(0, N_BLK)
    def blk(j):
        r0 = j * L
        row_idx = iota + r0
        cols = [jnp.maximum(
                    plsc.load_gather(rows_v,
                                     [row_idx, jnp.full((L,), d, jnp.int32)]),
                    0.0)
                for d in range(D_EMB)]
        ig_blk = ig_v[pl.ds(r0, L)] * D_OUT
        gg_blk = (gg_v[pl.ds(r0, L)] + N_IDX) * D_OUT
        obase = (iota + r0) * D_OUT
        for k in range(D_OUT):
            bias = (plsc.load_gather(wot_v, [ig_blk + k])
                    + plsc.load_gather(wot_v, [gg_blk + k]))
            terms = [cols[d] * wbc_v[pl.ds((k * D_EMB + d) * L, L)]
                     for d in range(D_EMB)] + [bias]
            while len(terms) > 1:
                terms = [terms[i] + terms[i + 1] if i + 1 < len(terms)
                         else terms[i] for i in range(0, len(terms), 2)]
            plsc.store_scatter(out_v, [obase + k], terms[0])
    pltpu.sync_copy(out_v, out_hbm.at[pl.ds(base * D_OUT, BPW * D_OUT)])


_run = pl.kernel(
    _tower_body,
    out_type=jax.ShapeDtypeStruct((BATCH * D_OUT,), jnp.float32),
    mesh=plsc.VectorSubcoreMesh(core_axis_name="c", subcore_axis_name="s"),
    compiler_params=pltpu.CompilerParams(needs_layout_passes=False,
                                         use_tc_tiling_on_sc=False),
    scratch_types=[
        pltpu.VMEM((N_CHUNK, CHUNK), jnp.int32),
        pltpu.VMEM((BPW, D_EMB), jnp.float32),
        pltpu.VMEM((D_EMB * BPW,), jnp.float32),
        pltpu.VMEM((BPW,), jnp.int32),
        pltpu.VMEM((BPW,), jnp.int32),
        pltpu.VMEM(((N_IDX + N_GAR) * D_OUT,), jnp.float32),
        pltpu.VMEM((D_OUT * D_EMB * L,), jnp.float32),
        pltpu.VMEM((BPW * D_OUT,), jnp.float32),
        pltpu.SemaphoreType.DMA,
    ],
)


def kernel(item_ids, index_group_names, garment_group_names, emb_table, W, b):
    ids = item_ids.astype(jnp.int32).reshape(NW, N_CHUNK, CHUNK)
    wbc = jnp.repeat(W[:, :D_EMB].reshape(-1), L)
    wot = jnp.concatenate(
        [W[:, D_EMB:D_EMB + N_IDX].T, W[:, D_EMB + N_IDX:].T + b[None, :]],
        axis=0,
    ).reshape(-1)
    out = _run(ids, index_group_names.astype(jnp.int32),
               garment_group_names.astype(jnp.int32), emb_table, wbc, wot)
    return out.reshape(BATCH, D_OUT)
